# baseline (speedup 1.0000x reference)
import functools

import jax
import jax.numpy as jnp
from jax import lax
from jax.experimental import pallas as pl
from jax.experimental.pallas import tpu as pltpu
from jax.experimental.pallas import tpu_sc as plsc

NC = 2
NS = 16
NW = NC * NS


def _sc_count(ei_flat, n_edges, n_pad, epw):
    mesh = plsc.VectorSubcoreMesh(core_axis_name="c", subcore_axis_name="s")

    @functools.partial(
        pl.kernel,
        out_type=jax.ShapeDtypeStruct((NW, n_pad), jnp.float32),
        mesh=mesh,
        compiler_params=pltpu.CompilerParams(needs_layout_passes=False),
        scratch_types=[
            pltpu.VMEM((epw,), jnp.int32),
            pltpu.VMEM((n_pad,), jnp.float32),
        ],
    )
    def k(ei_hbm, out_hbm, dst_v, cnt_v):
        wid = lax.axis_index("c") * NS + lax.axis_index("s")
        pltpu.sync_copy(ei_hbm.at[pl.ds(n_edges + wid * epw, epw)], dst_v)

        @plsc.parallel_loop(0, n_pad, 16, unroll=8)
        def zero_body(j):
            cnt_v[pl.ds(j, 16)] = jnp.zeros((16,), jnp.float32)

        ones = jnp.ones((16,), jnp.float32)

        @plsc.parallel_loop(0, epw, 16, unroll=2)
        def body(i):
            d = dst_v[pl.ds(i, 16)]
            plsc.addupdate_scatter(cnt_v, [d], ones)

        pltpu.sync_copy(cnt_v, out_hbm.at[wid])

    return k(ei_flat)


def _tc_z(x, w1t, n_pad):
    n = x.shape[0]
    h = w1t.shape[0]

    def body(x_ref, w1t_ref, z0_ref, z1_ref):
        zr = lax.dot_general(w1t_ref[...], x_ref[...],
                             (((1,), (1,)), ((), ())),
                             preferred_element_type=jnp.float32)
        for ref in (z0_ref, z1_ref):
            ref[...] = jnp.zeros((n_pad,), jnp.float32)
        z0_ref[pl.ds(0, n)] = zr[0]
        z1_ref[pl.ds(0, n)] = zr[1]

    assert h == 2
    return pl.pallas_call(
        body,
        out_shape=(
            jax.ShapeDtypeStruct((n_pad,), jnp.float32),
            jax.ShapeDtypeStruct((n_pad,), jnp.float32),
        ),
    )(x, w1t)


def _tc_dinv(cnt_parts):
    n_pad = cnt_parts.shape[1]

    def body(cnt_ref, dinv_ref):
        deg = jnp.sum(cnt_ref[...], axis=0) + 1.0
        dinv_ref[...] = lax.rsqrt(deg)

    return pl.pallas_call(
        body,
        out_shape=jax.ShapeDtypeStruct((n_pad,), jnp.float32),
    )(cnt_parts)


def _sc_main(ei_flat, z0, z1, dinv, n_edges, n_pad, epw):
    mesh = plsc.VectorSubcoreMesh(core_axis_name="c", subcore_axis_name="s")
    npw = n_pad // NW

    @functools.partial(
        pl.kernel,
        out_type=jax.ShapeDtypeStruct((3, NW, n_pad), jnp.float32),
        mesh=mesh,
        compiler_params=pltpu.CompilerParams(needs_layout_passes=False),
        scratch_types=[
            pltpu.VMEM((epw,), jnp.int32),
            pltpu.VMEM((epw,), jnp.int32),
            pltpu.VMEM((n_pad,), jnp.float32),
            pltpu.VMEM((n_pad,), jnp.float32),
            pltpu.VMEM((n_pad,), jnp.float32),
            pltpu.VMEM((n_pad,), jnp.float32),
            pltpu.VMEM((n_pad,), jnp.float32),
            pltpu.VMEM((n_pad,), jnp.float32),
        ],
    )
    def k(ei_hbm, z0_hbm, z1_hbm, dinv_hbm, out_hbm,
          src_v, dst_v, z0_v, z1_v, dinv_v, a0_v, a1_v, s_v):
        wid = lax.axis_index("c") * NS + lax.axis_index("s")
        pltpu.sync_copy(ei_hbm.at[pl.ds(wid * epw, epw)], src_v)
        pltpu.sync_copy(ei_hbm.at[pl.ds(n_edges + wid * epw, epw)], dst_v)
        pltpu.sync_copy(z0_hbm, z0_v)
        pltpu.sync_copy(z1_hbm, z1_v)
        pltpu.sync_copy(dinv_hbm, dinv_v)

        @plsc.parallel_loop(0, n_pad, 16, unroll=8)
        def zero_body(j):
            zz = jnp.zeros((16,), jnp.float32)
            a0_v[pl.ds(j, 16)] = zz
            a1_v[pl.ds(j, 16)] = zz
            s_v[pl.ds(j, 16)] = zz

        @plsc.parallel_loop(0, epw, 16, unroll=25)
        def body(off):
            sidx = src_v[pl.ds(off, 16)]
            didx = dst_v[pl.ds(off, 16)]
            dsv = plsc.load_gather(dinv_v, [sidx])
            ddv = plsc.load_gather(dinv_v, [didx])
            g0 = plsc.load_gather(z0_v, [sidx])
            g1 = plsc.load_gather(z1_v, [sidx])
            plsc.addupdate_scatter(a0_v, [didx], g0 * dsv)
            plsc.addupdate_scatter(a1_v, [didx], g1 * dsv)
            plsc.addupdate_scatter(s_v, [sidx], ddv)

        base = wid * npw

        @plsc.parallel_loop(base, base + npw, 16, unroll=4)
        def self_body(off):
            dv = dinv_v[pl.ds(off, 16)]
            g0 = z0_v[pl.ds(off, 16)]
            g1 = z1_v[pl.ds(off, 16)]
            a0_v[pl.ds(off, 16)] += g0 * dv
            a1_v[pl.ds(off, 16)] += g1 * dv
            s_v[pl.ds(off, 16)] += dv

        pltpu.sync_copy(a0_v, out_hbm.at[0, wid])
        pltpu.sync_copy(a1_v, out_hbm.at[1, wid])
        pltpu.sync_copy(s_v, out_hbm.at[2, wid])

    return k(ei_flat, z0, z1, dinv)


def _tc_final(parts, dinv, b1, W2, b2, Wout, bout_row, n_nodes):
    n_pad = parts.shape[2]
    n_out = Wout.shape[1]

    def body(parts_ref, dinv_ref, b1_ref, w2_ref, b2_ref, wout_ref, bout_ref,
             out_ref):
        a0 = jnp.sum(parts_ref[0], axis=0)
        a1 = jnp.sum(parts_ref[1], axis=0)
        s = jnp.sum(parts_ref[2], axis=0)
        dinv = dinv_ref[...]
        h10 = dinv * a0 + b1_ref[0]
        h11 = dinv * a1 + b1_ref[1]
        w = dinv * s
        mask = lax.broadcasted_iota(jnp.int32, (n_pad,), 0) < n_nodes
        t0 = jnp.sum(jnp.where(mask, h10 * w, 0.0))
        t1 = jnp.sum(jnp.where(mask, h11 * w, 0.0))
        inv_n = 1.0 / n_nodes
        p0 = (t0 * w2_ref[0, 0] + t1 * w2_ref[1, 0]) * inv_n + b2_ref[0]
        p1 = (t0 * w2_ref[0, 1] + t1 * w2_ref[1, 1]) * inv_n + b2_ref[1]
        out_ref[...] = (p0 * wout_ref[0:1, :] + p1 * wout_ref[1:2, :]
                        + bout_ref[...])

    smem = pl.BlockSpec(memory_space=pltpu.SMEM)
    return pl.pallas_call(
        body,
        in_specs=[pl.BlockSpec(), pl.BlockSpec(), smem, smem, smem,
                  pl.BlockSpec(), pl.BlockSpec()],
        out_shape=jax.ShapeDtypeStruct((1, n_out), jnp.float32),
    )(parts, dinv, b1, W2, b2, Wout, bout_row)


def kernel(x, edge_index, W1, b1, W2, b2, Wout, bout):
    n_nodes, d_feat = x.shape
    n_edges = edge_index.shape[1]

    grain = NW * 16
    n_pad = ((n_nodes + grain - 1) // grain) * grain
    epw = ((n_edges + NW * 16 - 1) // (NW * 16)) * 16
    assert epw * NW == n_edges, "edge padding not implemented for this shape"

    ei = edge_index
    if ei.dtype != jnp.int32:
        ei = ei.astype(jnp.int32)
    ei_flat = ei.reshape(-1)

    cnt = _sc_count(ei_flat, n_edges, n_pad, epw)
    z0, z1 = _tc_z(x, W1.T, n_pad)
    dinv = _tc_dinv(cnt)
    parts = _sc_main(ei_flat, z0, z1, dinv, n_edges, n_pad, epw)
    logits = _tc_final(parts, dinv, b1, W2, b2, Wout,
                       bout.reshape(1, -1), n_nodes)
    return logits


# speedup vs baseline: 1.0967x; 1.0967x over previous
import functools

import jax
import jax.numpy as jnp
from jax import lax
from jax.experimental import pallas as pl
from jax.experimental.pallas import tpu as pltpu
from jax.experimental.pallas import tpu_sc as plsc

NC = 2
NS = 16
NW = NC * NS
BLK = 128


def _sc_count(ei, n_pad, epw, rem):
    n_edges = ei.shape[1]
    mesh = plsc.VectorSubcoreMesh(core_axis_name="c", subcore_axis_name="s")

    @functools.partial(
        pl.kernel,
        out_type=jax.ShapeDtypeStruct((NW, n_pad), jnp.float32),
        mesh=mesh,
        compiler_params=pltpu.CompilerParams(needs_layout_passes=False),
        scratch_types=[
            pltpu.VMEM((2, epw), jnp.int32),
            pltpu.VMEM((2, BLK), jnp.int32),
            pltpu.VMEM((n_pad,), jnp.float32),
        ],
    )
    def k(ei_hbm, out_hbm, ed_v, ex_v, cnt_v):
        wid = lax.axis_index("c") * NS + lax.axis_index("s")
        pltpu.sync_copy(ei_hbm.at[:, pl.ds(wid * epw, epw)], ed_v)

        @plsc.parallel_loop(0, n_pad, 16, unroll=8)
        def zero_body(j):
            cnt_v[pl.ds(j, 16)] = jnp.zeros((16,), jnp.float32)

        ones = jnp.ones((16,), jnp.float32)

        @plsc.parallel_loop(0, epw, 16, unroll=4)
        def body(i):
            d = ed_v[1, pl.ds(i, 16)]
            plsc.addupdate_scatter(cnt_v, [d], ones)

        if rem:
            @pl.when(wid < rem)
            def _extra():
                base = n_edges - rem * BLK
                pltpu.sync_copy(ei_hbm.at[:, pl.ds(base + wid * BLK, BLK)],
                                ex_v)
                for j in range(BLK // 16):
                    d = ex_v[1, pl.ds(j * 16, 16)]
                    plsc.addupdate_scatter(cnt_v, [d], ones)

        pltpu.sync_copy(cnt_v, out_hbm.at[wid])

    return k(ei)


def _tc_z(x, w1t, n_pad):
    n = x.shape[0]
    h = w1t.shape[0]

    def body(x_ref, w1t_ref, z0_ref, z1_ref):
        zr = lax.dot_general(w1t_ref[...], x_ref[...],
                             (((1,), (1,)), ((), ())),
                             preferred_element_type=jnp.float32)
        for ref in (z0_ref, z1_ref):
            ref[...] = jnp.zeros((n_pad,), jnp.float32)
        z0_ref[pl.ds(0, n)] = zr[0]
        z1_ref[pl.ds(0, n)] = zr[1]

    assert h == 2
    return pl.pallas_call(
        body,
        out_shape=(
            jax.ShapeDtypeStruct((n_pad,), jnp.float32),
            jax.ShapeDtypeStruct((n_pad,), jnp.float32),
        ),
    )(x, w1t)


def _tc_dinv(cnt_parts):
    n_pad = cnt_parts.shape[1]

    def body(cnt_ref, dinv_ref):
        deg = jnp.sum(cnt_ref[...], axis=0) + 1.0
        dinv_ref[...] = lax.rsqrt(deg)

    return pl.pallas_call(
        body,
        out_shape=jax.ShapeDtypeStruct((n_pad,), jnp.float32),
    )(cnt_parts)


def _sc_main(ei, z0, z1, dinv, n_pad, epw, rem):
    n_edges = ei.shape[1]
    mesh = plsc.VectorSubcoreMesh(core_axis_name="c", subcore_axis_name="s")
    npw = n_pad // NW

    @functools.partial(
        pl.kernel,
        out_type=jax.ShapeDtypeStruct((3, NW, n_pad), jnp.float32),
        mesh=mesh,
        compiler_params=pltpu.CompilerParams(needs_layout_passes=False),
        scratch_types=[
            pltpu.VMEM((2, epw), jnp.int32),
            pltpu.VMEM((2, BLK), jnp.int32),
            pltpu.VMEM((n_pad,), jnp.float32),
            pltpu.VMEM((n_pad,), jnp.float32),
            pltpu.VMEM((n_pad,), jnp.float32),
            pltpu.VMEM((n_pad,), jnp.float32),
            pltpu.VMEM((n_pad,), jnp.float32),
            pltpu.VMEM((n_pad,), jnp.float32),
        ],
    )
    def k(ei_hbm, z0_hbm, z1_hbm, dinv_hbm, out_hbm,
          ed_v, ex_v, z0_v, z1_v, dinv_v, a0_v, a1_v, s_v):
        wid = lax.axis_index("c") * NS + lax.axis_index("s")
        pltpu.sync_copy(ei_hbm.at[:, pl.ds(wid * epw, epw)], ed_v)
        pltpu.sync_copy(z0_hbm, z0_v)
        pltpu.sync_copy(z1_hbm, z1_v)
        pltpu.sync_copy(dinv_hbm, dinv_v)

        @plsc.parallel_loop(0, n_pad, 16, unroll=8)
        def zero_body(j):
            zz = jnp.zeros((16,), jnp.float32)
            a0_v[pl.ds(j, 16)] = zz
            a1_v[pl.ds(j, 16)] = zz
            s_v[pl.ds(j, 16)] = zz

        def edge_chunk(ref, off):
            sidx = ref[0, pl.ds(off, 16)]
            didx = ref[1, pl.ds(off, 16)]
            dsv = plsc.load_gather(dinv_v, [sidx])
            ddv = plsc.load_gather(dinv_v, [didx])
            g0 = plsc.load_gather(z0_v, [sidx])
            g1 = plsc.load_gather(z1_v, [sidx])
            plsc.addupdate_scatter(a0_v, [didx], g0 * dsv)
            plsc.addupdate_scatter(a1_v, [didx], g1 * dsv)
            plsc.addupdate_scatter(s_v, [sidx], ddv)

        @plsc.parallel_loop(0, epw, 16, unroll=4)
        def body(off):
            edge_chunk(ed_v, off)

        if rem:
            @pl.when(wid < rem)
            def _extra():
                base = n_edges - rem * BLK
                pltpu.sync_copy(ei_hbm.at[:, pl.ds(base + wid * BLK, BLK)],
                                ex_v)
                for j in range(BLK // 16):
                    edge_chunk(ex_v, j * 16)

        base = wid * npw

        @plsc.parallel_loop(base, base + npw, 16, unroll=4)
        def self_body(off):
            dv = dinv_v[pl.ds(off, 16)]
            g0 = z0_v[pl.ds(off, 16)]
            g1 = z1_v[pl.ds(off, 16)]
            a0_v[pl.ds(off, 16)] += g0 * dv
            a1_v[pl.ds(off, 16)] += g1 * dv
            s_v[pl.ds(off, 16)] += dv

        pltpu.sync_copy(a0_v, out_hbm.at[0, wid])
        pltpu.sync_copy(a1_v, out_hbm.at[1, wid])
        pltpu.sync_copy(s_v, out_hbm.at[2, wid])

    return k(ei, z0, z1, dinv)


def _tc_final(parts, dinv, b1, W2, b2, Wout, bout_row, n_nodes):
    n_pad = parts.shape[2]
    n_out = Wout.shape[1]

    def body(parts_ref, dinv_ref, b1_ref, w2_ref, b2_ref, wout_ref, bout_ref,
             out_ref):
        a0 = jnp.sum(parts_ref[0], axis=0)
        a1 = jnp.sum(parts_ref[1], axis=0)
        s = jnp.sum(parts_ref[2], axis=0)
        dinv = dinv_ref[...]
        h10 = dinv * a0 + b1_ref[0]
        h11 = dinv * a1 + b1_ref[1]
        w = dinv * s
        mask = lax.broadcasted_iota(jnp.int32, (n_pad,), 0) < n_nodes
        t0 = jnp.sum(jnp.where(mask, h10 * w, 0.0))
        t1 = jnp.sum(jnp.where(mask, h11 * w, 0.0))
        inv_n = 1.0 / n_nodes
        p0 = (t0 * w2_ref[0, 0] + t1 * w2_ref[1, 0]) * inv_n + b2_ref[0]
        p1 = (t0 * w2_ref[0, 1] + t1 * w2_ref[1, 1]) * inv_n + b2_ref[1]
        out_ref[...] = (p0 * wout_ref[0:1, :] + p1 * wout_ref[1:2, :]
                        + bout_ref[...])

    smem = pl.BlockSpec(memory_space=pltpu.SMEM)
    return pl.pallas_call(
        body,
        in_specs=[pl.BlockSpec(), pl.BlockSpec(), smem, smem, smem,
                  pl.BlockSpec(), pl.BlockSpec()],
        out_shape=jax.ShapeDtypeStruct((1, n_out), jnp.float32),
    )(parts, dinv, b1, W2, b2, Wout, bout_row)


def kernel(x, edge_index, W1, b1, W2, b2, Wout, bout):
    n_nodes, d_feat = x.shape
    n_edges = edge_index.shape[1]

    grain = NW * 16
    n_pad = ((n_nodes + grain - 1) // grain) * grain
    assert n_edges % BLK == 0, "edge count must be a multiple of 128"
    nblk = n_edges // BLK
    epw = (nblk // NW) * BLK
    rem = nblk - (nblk // NW) * NW

    ei = edge_index
    if ei.dtype != jnp.int32:
        ei = ei.astype(jnp.int32)

    cnt = _sc_count(ei, n_pad, epw, rem)
    z0, z1 = _tc_z(x, W1.T, n_pad)
    dinv = _tc_dinv(cnt)
    parts = _sc_main(ei, z0, z1, dinv, n_pad, epw, rem)
    logits = _tc_final(parts, dinv, b1, W2, b2, Wout,
                       bout.reshape(1, -1), n_nodes)
    return logits


# speedup vs baseline: 1.1738x; 1.0703x over previous
import functools

import jax
import jax.numpy as jnp
from jax import lax
from jax.experimental import pallas as pl
from jax.experimental.pallas import tpu as pltpu
from jax.experimental.pallas import tpu_sc as plsc

NC = 2
NS = 16
NW = NC * NS
BLK = 128


def _sc_count(ei, n_pad, epw, rem):
    n_edges = ei.shape[1]
    mesh = plsc.VectorSubcoreMesh(core_axis_name="c", subcore_axis_name="s")

    @functools.partial(
        pl.kernel,
        out_type=jax.ShapeDtypeStruct((NW, n_pad), jnp.float32),
        mesh=mesh,
        compiler_params=pltpu.CompilerParams(needs_layout_passes=False),
        scratch_types=[
            pltpu.VMEM((2, epw), jnp.int32),
            pltpu.VMEM((2, BLK), jnp.int32),
            pltpu.VMEM((n_pad,), jnp.float32),
            pltpu.SemaphoreType.DMA,
        ],
    )
    def k(ei_hbm, out_hbm, ed_v, ex_v, cnt_v, sem):
        wid = lax.axis_index("c") * NS + lax.axis_index("s")
        cp = pltpu.async_copy(ei_hbm.at[:, pl.ds(wid * epw, epw)], ed_v, sem)

        @plsc.parallel_loop(0, n_pad, 16, unroll=8)
        def zero_body(j):
            cnt_v[pl.ds(j, 16)] = jnp.zeros((16,), jnp.float32)

        cp.wait()
        ones = jnp.ones((16,), jnp.float32)

        @plsc.parallel_loop(0, epw, 16, unroll=8)
        def body(i):
            d = ed_v[1, pl.ds(i, 16)]
            plsc.addupdate_scatter(cnt_v, [d], ones)

        if rem:
            @pl.when(wid < rem)
            def _extra():
                base = n_edges - rem * BLK
                pltpu.sync_copy(ei_hbm.at[:, pl.ds(base + wid * BLK, BLK)],
                                ex_v)
                for j in range(BLK // 16):
                    d = ex_v[1, pl.ds(j * 16, 16)]
                    plsc.addupdate_scatter(cnt_v, [d], ones)

        pltpu.sync_copy(cnt_v, out_hbm.at[wid])

    return k(ei)


def _tc_z(x, w1t, n_pad):
    n = x.shape[0]
    h = w1t.shape[0]

    def body(x_ref, w1t_ref, z0_ref, z1_ref):
        zr = lax.dot_general(w1t_ref[...], x_ref[...],
                             (((1,), (1,)), ((), ())),
                             preferred_element_type=jnp.float32)
        for ref in (z0_ref, z1_ref):
            ref[...] = jnp.zeros((n_pad,), jnp.float32)
        z0_ref[pl.ds(0, n)] = zr[0]
        z1_ref[pl.ds(0, n)] = zr[1]

    assert h == 2
    return pl.pallas_call(
        body,
        out_shape=(
            jax.ShapeDtypeStruct((n_pad,), jnp.float32),
            jax.ShapeDtypeStruct((n_pad,), jnp.float32),
        ),
    )(x, w1t)


def _tc_dinv(cnt_parts):
    n_pad = cnt_parts.shape[1]

    def body(cnt_ref, dinv_ref):
        deg = jnp.sum(cnt_ref[...], axis=0) + 1.0
        dinv_ref[...] = lax.rsqrt(deg)

    return pl.pallas_call(
        body,
        out_shape=jax.ShapeDtypeStruct((n_pad,), jnp.float32),
    )(cnt_parts)


def _sc_main(ei, z0, z1, dinv, n_pad, epw, rem):
    n_edges = ei.shape[1]
    mesh = plsc.VectorSubcoreMesh(core_axis_name="c", subcore_axis_name="s")
    npw = n_pad // NW

    @functools.partial(
        pl.kernel,
        out_type=jax.ShapeDtypeStruct((3, NW, n_pad), jnp.float32),
        mesh=mesh,
        compiler_params=pltpu.CompilerParams(needs_layout_passes=False),
        scratch_types=[
            pltpu.VMEM((2, epw), jnp.int32),
            pltpu.VMEM((2, BLK), jnp.int32),
            pltpu.VMEM((n_pad,), jnp.float32),
            pltpu.VMEM((n_pad,), jnp.float32),
            pltpu.VMEM((n_pad,), jnp.float32),
            pltpu.VMEM((n_pad,), jnp.float32),
            pltpu.VMEM((n_pad,), jnp.float32),
            pltpu.VMEM((n_pad,), jnp.float32),
            pltpu.SemaphoreType.DMA,
            pltpu.SemaphoreType.DMA,
        ],
    )
    def k(ei_hbm, z0_hbm, z1_hbm, dinv_hbm, out_hbm,
          ed_v, ex_v, z0_v, z1_v, dinv_v, a0_v, a1_v, s_v, sem_e, sem_t):
        wid = lax.axis_index("c") * NS + lax.axis_index("s")
        cp_e = pltpu.async_copy(ei_hbm.at[:, pl.ds(wid * epw, epw)], ed_v,
                                sem_e)
        cp_z0 = pltpu.async_copy(z0_hbm, z0_v, sem_t)
        cp_z1 = pltpu.async_copy(z1_hbm, z1_v, sem_t)
        cp_d = pltpu.async_copy(dinv_hbm, dinv_v, sem_t)

        @plsc.parallel_loop(0, n_pad, 16, unroll=8)
        def zero_body(j):
            zz = jnp.zeros((16,), jnp.float32)
            a0_v[pl.ds(j, 16)] = zz
            a1_v[pl.ds(j, 16)] = zz
            s_v[pl.ds(j, 16)] = zz

        cp_e.wait()
        cp_z0.wait()
        cp_z1.wait()
        cp_d.wait()

        def edge_chunk(ref, off):
            sidx = ref[0, pl.ds(off, 16)]
            didx = ref[1, pl.ds(off, 16)]
            dsv = plsc.load_gather(dinv_v, [sidx])
            ddv = plsc.load_gather(dinv_v, [didx])
            g0 = plsc.load_gather(z0_v, [sidx])
            g1 = plsc.load_gather(z1_v, [sidx])
            plsc.addupdate_scatter(a0_v, [didx], g0 * dsv)
            plsc.addupdate_scatter(a1_v, [didx], g1 * dsv)
            plsc.addupdate_scatter(s_v, [sidx], ddv)

        @plsc.parallel_loop(0, epw, 16, unroll=8)
        def body(off):
            edge_chunk(ed_v, off)

        if rem:
            @pl.when(wid < rem)
            def _extra():
                base = n_edges - rem * BLK
                pltpu.sync_copy(ei_hbm.at[:, pl.ds(base + wid * BLK, BLK)],
                                ex_v)
                for j in range(BLK // 16):
                    edge_chunk(ex_v, j * 16)

        base = wid * npw

        @plsc.parallel_loop(base, base + npw, 16, unroll=4)
        def self_body(off):
            dv = dinv_v[pl.ds(off, 16)]
            g0 = z0_v[pl.ds(off, 16)]
            g1 = z1_v[pl.ds(off, 16)]
            a0_v[pl.ds(off, 16)] += g0 * dv
            a1_v[pl.ds(off, 16)] += g1 * dv
            s_v[pl.ds(off, 16)] += dv

        pltpu.sync_copy(a0_v, out_hbm.at[0, wid])
        pltpu.sync_copy(a1_v, out_hbm.at[1, wid])
        pltpu.sync_copy(s_v, out_hbm.at[2, wid])

    return k(ei, z0, z1, dinv)


def _tc_final(parts, dinv, b1, W2, b2, Wout, bout_row, n_nodes):
    n_pad = parts.shape[2]
    n_out = Wout.shape[1]

    def body(parts_ref, dinv_ref, b1_ref, w2_ref, b2_ref, wout_ref, bout_ref,
             out_ref):
        a0 = jnp.sum(parts_ref[0], axis=0)
        a1 = jnp.sum(parts_ref[1], axis=0)
        s = jnp.sum(parts_ref[2], axis=0)
        dinv = dinv_ref[...]
        h10 = dinv * a0 + b1_ref[0]
        h11 = dinv * a1 + b1_ref[1]
        w = dinv * s
        mask = lax.broadcasted_iota(jnp.int32, (n_pad,), 0) < n_nodes
        t0 = jnp.sum(jnp.where(mask, h10 * w, 0.0))
        t1 = jnp.sum(jnp.where(mask, h11 * w, 0.0))
        inv_n = 1.0 / n_nodes
        p0 = (t0 * w2_ref[0, 0] + t1 * w2_ref[1, 0]) * inv_n + b2_ref[0]
        p1 = (t0 * w2_ref[0, 1] + t1 * w2_ref[1, 1]) * inv_n + b2_ref[1]
        out_ref[...] = (p0 * wout_ref[0:1, :] + p1 * wout_ref[1:2, :]
                        + bout_ref[...])

    smem = pl.BlockSpec(memory_space=pltpu.SMEM)
    return pl.pallas_call(
        body,
        in_specs=[pl.BlockSpec(), pl.BlockSpec(), smem, smem, smem,
                  pl.BlockSpec(), pl.BlockSpec()],
        out_shape=jax.ShapeDtypeStruct((1, n_out), jnp.float32),
    )(parts, dinv, b1, W2, b2, Wout, bout_row)


def kernel(x, edge_index, W1, b1, W2, b2, Wout, bout):
    n_nodes, d_feat = x.shape
    n_edges = edge_index.shape[1]

    grain = NW * 16
    n_pad = ((n_nodes + grain - 1) // grain) * grain
    assert n_edges % BLK == 0, "edge count must be a multiple of 128"
    nblk = n_edges // BLK
    epw = (nblk // NW) * BLK
    rem = nblk - (nblk // NW) * NW

    ei = edge_index
    if ei.dtype != jnp.int32:
        ei = ei.astype(jnp.int32)

    cnt = _sc_count(ei, n_pad, epw, rem)
    z0, z1 = _tc_z(x, W1.T, n_pad)
    dinv = _tc_dinv(cnt)
    parts = _sc_main(ei, z0, z1, dinv, n_pad, epw, rem)
    logits = _tc_final(parts, dinv, b1, W2, b2, Wout,
                       bout.reshape(1, -1), n_nodes)
    return logits


# speedup vs baseline: 1.1766x; 1.0024x over previous
import functools

import jax
import jax.numpy as jnp
from jax import lax
from jax.experimental import pallas as pl
from jax.experimental.pallas import tpu as pltpu
from jax.experimental.pallas import tpu_sc as plsc

NC = 2
NS = 16
NW = NC * NS
BLK = 128


def _sc_count(ei, n_pad, epw, rem):
    n_edges = ei.shape[1]
    mesh = plsc.VectorSubcoreMesh(core_axis_name="c", subcore_axis_name="s")

    @functools.partial(
        pl.kernel,
        out_type=jax.ShapeDtypeStruct((NW, n_pad), jnp.float32),
        mesh=mesh,
        compiler_params=pltpu.CompilerParams(needs_layout_passes=False),
        scratch_types=[
            pltpu.VMEM((2, epw), jnp.int32),
            pltpu.VMEM((2, BLK), jnp.int32),
            pltpu.VMEM((n_pad,), jnp.float32),
            pltpu.SemaphoreType.DMA,
        ],
    )
    def k(ei_hbm, out_hbm, ed_v, ex_v, cnt_v, sem):
        wid = lax.axis_index("c") * NS + lax.axis_index("s")
        cp = pltpu.async_copy(ei_hbm.at[:, pl.ds(wid * epw, epw)], ed_v, sem)

        @plsc.parallel_loop(0, n_pad, 16, unroll=8)
        def zero_body(j):
            cnt_v[pl.ds(j, 16)] = jnp.zeros((16,), jnp.float32)

        cp.wait()
        ones = jnp.ones((16,), jnp.float32)

        @plsc.parallel_loop(0, epw, 16, unroll=8)
        def body(i):
            d = ed_v[1, pl.ds(i, 16)]
            plsc.addupdate_scatter(cnt_v, [d], ones)

        if rem:
            @pl.when(wid < rem)
            def _extra():
                base = n_edges - rem * BLK
                pltpu.sync_copy(ei_hbm.at[:, pl.ds(base + wid * BLK, BLK)],
                                ex_v)
                for j in range(BLK // 16):
                    d = ex_v[1, pl.ds(j * 16, 16)]
                    plsc.addupdate_scatter(cnt_v, [d], ones)

        pltpu.sync_copy(cnt_v, out_hbm.at[wid])

    return k(ei)


def _tc_z(x, w1t, n_pad):
    n = x.shape[0]
    h = w1t.shape[0]

    def body(x_ref, w1t_ref, z0_ref, z1_ref):
        zr = lax.dot_general(w1t_ref[...], x_ref[...],
                             (((1,), (1,)), ((), ())),
                             preferred_element_type=jnp.float32)
        for ref in (z0_ref, z1_ref):
            ref[...] = jnp.zeros((n_pad,), jnp.float32)
        z0_ref[pl.ds(0, n)] = zr[0]
        z1_ref[pl.ds(0, n)] = zr[1]

    assert h == 2
    return pl.pallas_call(
        body,
        out_shape=(
            jax.ShapeDtypeStruct((n_pad,), jnp.float32),
            jax.ShapeDtypeStruct((n_pad,), jnp.float32),
        ),
    )(x, w1t)


def _tc_dinv(cnt_parts):
    n_pad = cnt_parts.shape[1]

    def body(cnt_ref, dinv_ref):
        deg = jnp.sum(cnt_ref[...], axis=0) + 1.0
        dinv_ref[...] = lax.rsqrt(deg)

    return pl.pallas_call(
        body,
        out_shape=jax.ShapeDtypeStruct((n_pad,), jnp.float32),
    )(cnt_parts)


def _sc_main(ei, z0, z1, dinv, n_pad, epw, rem):
    n_edges = ei.shape[1]
    mesh = plsc.VectorSubcoreMesh(core_axis_name="c", subcore_axis_name="s")
    npw = n_pad // NW

    @functools.partial(
        pl.kernel,
        out_type=jax.ShapeDtypeStruct((3, NW, n_pad), jnp.float32),
        mesh=mesh,
        compiler_params=pltpu.CompilerParams(needs_layout_passes=False),
        scratch_types=[
            pltpu.VMEM((2, epw), jnp.int32),
            pltpu.VMEM((2, BLK), jnp.int32),
            pltpu.VMEM((n_pad,), jnp.float32),
            pltpu.VMEM((n_pad,), jnp.float32),
            pltpu.VMEM((n_pad,), jnp.float32),
            pltpu.VMEM((n_pad,), jnp.float32),
            pltpu.VMEM((n_pad,), jnp.float32),
            pltpu.VMEM((n_pad,), jnp.float32),
            pltpu.SemaphoreType.DMA,
            pltpu.SemaphoreType.DMA,
        ],
    )
    def k(ei_hbm, z0_hbm, z1_hbm, dinv_hbm, out_hbm,
          ed_v, ex_v, z0_v, z1_v, dinv_v, a0_v, a1_v, s_v, sem_e, sem_t):
        wid = lax.axis_index("c") * NS + lax.axis_index("s")
        cp_e = pltpu.async_copy(ei_hbm.at[:, pl.ds(wid * epw, epw)], ed_v,
                                sem_e)
        cp_z0 = pltpu.async_copy(z0_hbm, z0_v, sem_t)
        cp_z1 = pltpu.async_copy(z1_hbm, z1_v, sem_t)
        cp_d = pltpu.async_copy(dinv_hbm, dinv_v, sem_t)

        @plsc.parallel_loop(0, n_pad, 16, unroll=8)
        def zero_body(j):
            zz = jnp.zeros((16,), jnp.float32)
            a0_v[pl.ds(j, 16)] = zz
            a1_v[pl.ds(j, 16)] = zz
            s_v[pl.ds(j, 16)] = zz

        cp_e.wait()
        cp_z0.wait()
        cp_z1.wait()
        cp_d.wait()

        def edge_chunk(ref, off):
            sidx = ref[0, pl.ds(off, 16)]
            didx = ref[1, pl.ds(off, 16)]
            dsv = plsc.load_gather(dinv_v, [sidx])
            ddv = plsc.load_gather(dinv_v, [didx])
            g0 = plsc.load_gather(z0_v, [sidx])
            g1 = plsc.load_gather(z1_v, [sidx])
            plsc.addupdate_scatter(a0_v, [didx], g0 * dsv)
            plsc.addupdate_scatter(a1_v, [didx], g1 * dsv)
            plsc.addupdate_scatter(s_v, [sidx], ddv)

        @plsc.parallel_loop(0, epw, 16, unroll=16)
        def body(off):
            edge_chunk(ed_v, off)

        if rem:
            @pl.when(wid < rem)
            def _extra():
                base = n_edges - rem * BLK
                pltpu.sync_copy(ei_hbm.at[:, pl.ds(base + wid * BLK, BLK)],
                                ex_v)
                for j in range(BLK // 16):
                    edge_chunk(ex_v, j * 16)

        base = wid * npw

        @plsc.parallel_loop(base, base + npw, 16, unroll=4)
        def self_body(off):
            dv = dinv_v[pl.ds(off, 16)]
            g0 = z0_v[pl.ds(off, 16)]
            g1 = z1_v[pl.ds(off, 16)]
            a0_v[pl.ds(off, 16)] += g0 * dv
            a1_v[pl.ds(off, 16)] += g1 * dv
            s_v[pl.ds(off, 16)] += dv

        pltpu.sync_copy(a0_v, out_hbm.at[0, wid])
        pltpu.sync_copy(a1_v, out_hbm.at[1, wid])
        pltpu.sync_copy(s_v, out_hbm.at[2, wid])

    return k(ei, z0, z1, dinv)


def _tc_final(parts, dinv, b1, W2, b2, Wout, bout_row, n_nodes):
    n_pad = parts.shape[2]
    n_out = Wout.shape[1]

    def body(parts_ref, dinv_ref, b1_ref, w2_ref, b2_ref, wout_ref, bout_ref,
             out_ref):
        a0 = jnp.sum(parts_ref[0], axis=0)
        a1 = jnp.sum(parts_ref[1], axis=0)
        s = jnp.sum(parts_ref[2], axis=0)
        dinv = dinv_ref[...]
        h10 = dinv * a0 + b1_ref[0]
        h11 = dinv * a1 + b1_ref[1]
        w = dinv * s
        mask = lax.broadcasted_iota(jnp.int32, (n_pad,), 0) < n_nodes
        t0 = jnp.sum(jnp.where(mask, h10 * w, 0.0))
        t1 = jnp.sum(jnp.where(mask, h11 * w, 0.0))
        inv_n = 1.0 / n_nodes
        p0 = (t0 * w2_ref[0, 0] + t1 * w2_ref[1, 0]) * inv_n + b2_ref[0]
        p1 = (t0 * w2_ref[0, 1] + t1 * w2_ref[1, 1]) * inv_n + b2_ref[1]
        out_ref[...] = (p0 * wout_ref[0:1, :] + p1 * wout_ref[1:2, :]
                        + bout_ref[...])

    smem = pl.BlockSpec(memory_space=pltpu.SMEM)
    return pl.pallas_call(
        body,
        in_specs=[pl.BlockSpec(), pl.BlockSpec(), smem, smem, smem,
                  pl.BlockSpec(), pl.BlockSpec()],
        out_shape=jax.ShapeDtypeStruct((1, n_out), jnp.float32),
    )(parts, dinv, b1, W2, b2, Wout, bout_row)


def kernel(x, edge_index, W1, b1, W2, b2, Wout, bout):
    n_nodes, d_feat = x.shape
    n_edges = edge_index.shape[1]

    grain = NW * 16
    n_pad = ((n_nodes + grain - 1) // grain) * grain
    assert n_edges % BLK == 0, "edge count must be a multiple of 128"
    nblk = n_edges // BLK
    epw = (nblk // NW) * BLK
    rem = nblk - (nblk // NW) * NW

    ei = edge_index
    if ei.dtype != jnp.int32:
        ei = ei.astype(jnp.int32)

    cnt = _sc_count(ei, n_pad, epw, rem)
    z0, z1 = _tc_z(x, W1.T, n_pad)
    dinv = _tc_dinv(cnt)
    parts = _sc_main(ei, z0, z1, dinv, n_pad, epw, rem)
    logits = _tc_final(parts, dinv, b1, W2, b2, Wout,
                       bout.reshape(1, -1), n_nodes)
    return logits
